# baseline (speedup 1.0000x reference)
import functools

import jax
import jax.numpy as jnp
from jax.experimental import pallas as pl
from jax.experimental.pallas import tpu as pltpu

DIM = 4096
NUM_EXPERTS = 64
TOP_K = 8
TOKEN_BLOCK = 1024


def _gate_body(x_ref, wt_ref, w_out_ref, i_out_ref):
    logits = jax.lax.dot_general(
        x_ref[...], wt_ref[...],
        dimension_numbers=(((1,), (0,)), ((), ())),
        preferred_element_type=jnp.float32,
    )
    tb = logits.shape[0]
    lt = logits.T
    row = jax.lax.broadcasted_iota(jnp.int32, lt.shape, 0)
    row8 = jax.lax.broadcasted_iota(jnp.int32, (TOP_K, tb), 0)
    neg_inf = jnp.float32(float("-inf"))

    work = lt
    top_v = jnp.zeros((TOP_K, tb), jnp.float32)
    top_i = jnp.zeros((TOP_K, tb), jnp.int32)
    for k in range(TOP_K):
        m = jnp.max(work, axis=0, keepdims=True)
        idx = jnp.min(jnp.where(work == m, row, NUM_EXPERTS), axis=0,
                      keepdims=True)
        top_v = jnp.where(row8 == k, m, top_v)
        top_i = jnp.where(row8 == k, idx, top_i)
        work = jnp.where(row == idx, neg_inf, work)

    m0 = jnp.max(top_v, axis=0, keepdims=True)
    e = jnp.exp(top_v - m0)
    w8 = e / jnp.sum(e, axis=0, keepdims=True)

    eye8 = jnp.eye(TOP_K, dtype=jnp.float32)
    tdims = (((0,), (0,)), ((), ()))
    w_out_ref[...] = jax.lax.dot_general(
        w8, eye8, dimension_numbers=tdims,
        preferred_element_type=jnp.float32)
    i_out_ref[...] = jax.lax.dot_general(
        top_i.astype(jnp.float32), eye8, dimension_numbers=tdims,
        preferred_element_type=jnp.float32).astype(jnp.int32)


@functools.partial(jax.jit, static_argnames=("interpret",))
def kernel(x, W, interpret=False):
    b, n, d = x.shape
    tokens = b * n
    xt = x.reshape(tokens, d)
    wt = W.T
    grid = (tokens // TOKEN_BLOCK,)
    weights, indices = pl.pallas_call(
        _gate_body,
        grid=grid,
        in_specs=[
            pl.BlockSpec((TOKEN_BLOCK, d), lambda i: (i, 0)),
            pl.BlockSpec((d, NUM_EXPERTS), lambda i: (0, 0)),
        ],
        out_specs=[
            pl.BlockSpec((TOKEN_BLOCK, TOP_K), lambda i: (i, 0)),
            pl.BlockSpec((TOKEN_BLOCK, TOP_K), lambda i: (i, 0)),
        ],
        out_shape=[
            jax.ShapeDtypeStruct((tokens, TOP_K), jnp.float32),
            jax.ShapeDtypeStruct((tokens, TOP_K), jnp.int32),
        ],
        compiler_params=pltpu.CompilerParams(
            dimension_semantics=("arbitrary",),
        ),
        interpret=interpret,
    )(xt, wt)
    return weights.reshape(b, n, TOP_K), indices.reshape(b, n, TOP_K)


# speedup vs baseline: 1.1819x; 1.1819x over previous
import jax
import jax.numpy as jnp
from jax.experimental import pallas as pl
from jax.experimental.pallas import tpu as pltpu

DIM = 4096
NUM_EXPERTS = 64
TOP_K = 8
TOKEN_BLOCK = 1024


def _gate_body(x_ref, wt_ref, w_out_ref, i_out_ref):
    logits = jax.lax.dot_general(
        x_ref[...], wt_ref[...],
        dimension_numbers=(((1,), (0,)), ((), ())),
        preferred_element_type=jnp.float32,
    )
    tb = logits.shape[0]
    lt = logits.T
    row = jax.lax.broadcasted_iota(jnp.int32, lt.shape, 0)
    row8 = jax.lax.broadcasted_iota(jnp.int32, (TOP_K, tb), 0)
    neg_inf = jnp.float32(float("-inf"))

    work = lt
    top_v = jnp.zeros((TOP_K, tb), jnp.float32)
    top_i = jnp.zeros((TOP_K, tb), jnp.int32)
    for k in range(TOP_K):
        m = jnp.max(work, axis=0, keepdims=True)
        idx = jnp.min(jnp.where(work == m, row, NUM_EXPERTS), axis=0,
                      keepdims=True)
        top_v = jnp.where(row8 == k, m, top_v)
        top_i = jnp.where(row8 == k, idx, top_i)
        work = jnp.where(row == idx, neg_inf, work)

    m0 = jnp.max(top_v, axis=0, keepdims=True)
    e = jnp.exp(top_v - m0)
    w_out_ref[...] = e / jnp.sum(e, axis=0, keepdims=True)
    i_out_ref[...] = top_i


@jax.jit
def kernel(x, W):
    b, n, d = x.shape
    tokens = b * n
    xt = x.reshape(tokens, d)
    wt = W.T
    grid = (tokens // TOKEN_BLOCK,)
    weights, indices = pl.pallas_call(
        _gate_body,
        grid=grid,
        in_specs=[
            pl.BlockSpec((TOKEN_BLOCK, d), lambda i: (i, 0)),
            pl.BlockSpec((d, NUM_EXPERTS), lambda i: (0, 0)),
        ],
        out_specs=[
            pl.BlockSpec((TOP_K, TOKEN_BLOCK), lambda i: (0, i)),
            pl.BlockSpec((TOP_K, TOKEN_BLOCK), lambda i: (0, i)),
        ],
        out_shape=[
            jax.ShapeDtypeStruct((TOP_K, tokens), jnp.float32),
            jax.ShapeDtypeStruct((TOP_K, tokens), jnp.int32),
        ],
        compiler_params=pltpu.CompilerParams(
            dimension_semantics=("arbitrary",),
        ),
    )(xt, wt)
    return weights.T.reshape(b, n, TOP_K), indices.T.reshape(b, n, TOP_K)


# speedup vs baseline: 1.2319x; 1.0423x over previous
import jax
import jax.numpy as jnp
from jax.experimental import pallas as pl
from jax.experimental.pallas import tpu as pltpu

DIM = 4096
NUM_EXPERTS = 64
TOP_K = 8
TOKEN_BLOCK = 1024


def _gate_body(x_ref, wt_ref, w_out_ref, i_out_ref):
    logits = jax.lax.dot_general(
        x_ref[...], wt_ref[...],
        dimension_numbers=(((1,), (1,)), ((), ())),
        preferred_element_type=jnp.float32,
    )
    tb = logits.shape[0]
    lt = logits.T
    row = jax.lax.broadcasted_iota(jnp.int32, lt.shape, 0)
    row8 = jax.lax.broadcasted_iota(jnp.int32, (TOP_K, tb), 0)
    neg_inf = jnp.float32(float("-inf"))

    work = lt
    top_v = jnp.zeros((TOP_K, tb), jnp.float32)
    top_i = jnp.zeros((TOP_K, tb), jnp.int32)
    for k in range(TOP_K):
        m = jnp.max(work, axis=0, keepdims=True)
        idx = jnp.min(jnp.where(work == m, row, NUM_EXPERTS), axis=0,
                      keepdims=True)
        top_v = jnp.where(row8 == k, m, top_v)
        top_i = jnp.where(row8 == k, idx, top_i)
        work = jnp.where(row == idx, neg_inf, work)

    m0 = jnp.max(top_v, axis=0, keepdims=True)
    e = jnp.exp(top_v - m0)
    w_out_ref[...] = e / jnp.sum(e, axis=0, keepdims=True)
    i_out_ref[...] = top_i


@jax.jit
def kernel(x, W):
    b, n, d = x.shape
    tokens = b * n
    xt = x.reshape(tokens, d)
    grid = (tokens // TOKEN_BLOCK,)
    weights, indices = pl.pallas_call(
        _gate_body,
        grid=grid,
        in_specs=[
            pl.BlockSpec((TOKEN_BLOCK, d), lambda i: (i, 0)),
            pl.BlockSpec((NUM_EXPERTS, d), lambda i: (0, 0)),
        ],
        out_specs=[
            pl.BlockSpec((TOP_K, TOKEN_BLOCK), lambda i: (0, i)),
            pl.BlockSpec((TOP_K, TOKEN_BLOCK), lambda i: (0, i)),
        ],
        out_shape=[
            jax.ShapeDtypeStruct((TOP_K, tokens), jnp.float32),
            jax.ShapeDtypeStruct((TOP_K, tokens), jnp.int32),
        ],
        compiler_params=pltpu.CompilerParams(
            dimension_semantics=("arbitrary",),
        ),
    )(xt, W)
    return weights.T.reshape(b, n, TOP_K), indices.T.reshape(b, n, TOP_K)
